# Initial kernel scaffold; baseline (speedup 1.0000x reference)
import functools

import jax
import jax.numpy as jnp
from jax import lax
from jax.experimental import pallas as pl
from jax.experimental.pallas import tpu as pltpu
from jax.experimental.pallas import tpu_sc as plsc

N_NODES = 10000
IN_DIM = 128
HID_DIM = 128
LAT_DIM = 64

NC, NS = 2, 16
NW = NC * NS
K = 128
N_ACC = 10016
CNT_W = 16
RPS = N_ACC // NS


def _mesh():
    return plsc.VectorSubcoreMesh(core_axis_name="c", subcore_axis_name="s")


@functools.lru_cache(maxsize=None)
def _make_count(e_pad):
    e_w = e_pad // NW
    n_chunks = e_w // K

    @functools.partial(
        pl.kernel,
        out_type=jax.ShapeDtypeStruct((NC, N_ACC, CNT_W), jnp.float32),
        mesh=_mesh(),
        scratch_types=[
            pltpu.VMEM((2, K), jnp.int32),
            pltpu.VMEM((K, CNT_W), jnp.float32),
            pltpu.VMEM_SHARED((N_ACC, CNT_W), jnp.float32),
        ],
    )
    def count_kernel(dst_hbm, ones_hbm, zeros_hbm, out_hbm, idxb, onesb, acc):
        c = lax.axis_index("c")
        s = lax.axis_index("s")
        wid = c * NS + s
        pltpu.sync_copy(ones_hbm, onesb)
        pltpu.sync_copy(zeros_hbm.at[pl.ds(s * RPS, RPS)],
                        acc.at[pl.ds(s * RPS, RPS)])
        plsc.subcore_barrier()

        @pl.loop(0, n_chunks)
        def _(j):
            base = wid * e_w + j * K
            pltpu.sync_copy(dst_hbm.at[pl.ds(base, K)], idxb.at[0])
            pltpu.sync_copy(onesb, acc.at[idxb.at[0]], add=True)

        plsc.subcore_barrier()
        pltpu.sync_copy(acc.at[pl.ds(s * RPS, RPS)],
                        out_hbm.at[c].at[pl.ds(s * RPS, RPS)])

    return count_kernel


@functools.lru_cache(maxsize=None)
def _make_spmm(e_pad, d):
    e_w = e_pad // NW
    n_chunks = e_w // K

    @functools.partial(
        pl.kernel,
        out_type=jax.ShapeDtypeStruct((NC, N_ACC, d), jnp.float32),
        mesh=_mesh(),
        scratch_types=[
            pltpu.VMEM((2, K), jnp.int32),
            pltpu.VMEM((2, K), jnp.int32),
            pltpu.VMEM((K, d), jnp.float32),
            pltpu.VMEM_SHARED((N_ACC, d), jnp.float32),
            pltpu.SemaphoreType.DMA,
        ],
    )
    def spmm_kernel(xs_hbm, src_hbm, dst_hbm, zeros_hbm, out_hbm,
                    srcb, dstb, rowsb, acc, sem):
        c = lax.axis_index("c")
        s = lax.axis_index("s")
        wid = c * NS + s
        pltpu.sync_copy(zeros_hbm.at[pl.ds(s * RPS, RPS)],
                        acc.at[pl.ds(s * RPS, RPS)])
        plsc.subcore_barrier()

        @pl.loop(0, n_chunks)
        def _(j):
            base = wid * e_w + j * K
            pltpu.sync_copy(src_hbm.at[pl.ds(base, K)], srcb.at[0])
            pltpu.sync_copy(dst_hbm.at[pl.ds(base, K)], dstb.at[0])
            pltpu.async_copy(xs_hbm.at[srcb.at[0]], rowsb, sem).wait()
            pltpu.sync_copy(rowsb, acc.at[dstb.at[0]], add=True)

        plsc.subcore_barrier()
        pltpu.sync_copy(acc.at[pl.ds(s * RPS, RPS)],
                        out_hbm.at[c].at[pl.ds(s * RPS, RPS)])

    return spmm_kernel


def _tc_scale_xw(x, w, cnt0, cnt1):
    def body(x_ref, w_ref, c0_ref, c1_ref, o_ref):
        dinv = lax.rsqrt(c0_ref[...] + c1_ref[...] + 1.0)
        xw = jnp.dot(x_ref[...], w_ref[...], preferred_element_type=jnp.float32)
        o_ref[...] = xw * dinv

    return pl.pallas_call(
        body,
        out_shape=jax.ShapeDtypeStruct((x.shape[0], w.shape[1]), jnp.float32),
    )(x, w, cnt0, cnt1)


def _tc_combine_next(a0, a1, xs, cnt0, cnt1, b, w):
    def body(a0_ref, a1_ref, xs_ref, c0_ref, c1_ref, b_ref, w_ref, o_ref):
        dinv = lax.rsqrt(c0_ref[...] + c1_ref[...] + 1.0)
        h = dinv * (a0_ref[...] + a1_ref[...] + xs_ref[...]) + b_ref[...]
        h = jnp.maximum(h, 0.0)
        hw = jnp.dot(h, w_ref[...], preferred_element_type=jnp.float32)
        o_ref[...] = hw * dinv

    return pl.pallas_call(
        body,
        out_shape=jax.ShapeDtypeStruct((a0.shape[0], w.shape[1]), jnp.float32),
    )(a0, a1, xs, cnt0, cnt1, b, w)


def _tc_final(a0, a1, xs, cnt0, cnt1, b):
    def body(a0_ref, a1_ref, xs_ref, c0_ref, c1_ref, b_ref, o_ref):
        dinv = lax.rsqrt(c0_ref[...] + c1_ref[...] + 1.0)
        o_ref[...] = dinv * (a0_ref[...] + a1_ref[...] + xs_ref[...]) + b_ref[...]

    return pl.pallas_call(
        body,
        out_shape=jax.ShapeDtypeStruct(a0.shape, jnp.float32),
    )(a0, a1, xs, cnt0, cnt1, b)


def kernel(x, edge_index, W1, b1, W2, b2):
    n = x.shape[0]
    e = edge_index.shape[1]
    e_pad = ((e + NW * K - 1) // (NW * K)) * (NW * K)

    src = edge_index[0].astype(jnp.int32)
    dst = edge_index[1].astype(jnp.int32)
    pad = jnp.full((e_pad - e,), N_NODES, dtype=jnp.int32)
    src = jnp.concatenate([src, pad])
    dst = jnp.concatenate([dst, pad])

    ones_cnt = jnp.ones((K, CNT_W), jnp.float32)
    zeros_cnt = jnp.zeros((N_ACC, CNT_W), jnp.float32)
    zeros_h = jnp.zeros((N_ACC, HID_DIM), jnp.float32)
    zeros_l = jnp.zeros((N_ACC, LAT_DIM), jnp.float32)

    counts = _make_count(e_pad)(dst, ones_cnt, zeros_cnt)
    cnt0 = counts[0, :n, :1]
    cnt1 = counts[1, :n, :1]

    xs1 = _tc_scale_xw(x, W1, cnt0, cnt1)
    xs1_pad = jnp.pad(xs1, ((0, N_ACC - n), (0, 0)))

    acc1 = _make_spmm(e_pad, HID_DIM)(xs1_pad, src, dst, zeros_h)
    xs2 = _tc_combine_next(acc1[0, :n], acc1[1, :n], xs1, cnt0, cnt1,
                           b1.reshape(1, -1), W2)
    xs2_pad = jnp.pad(xs2, ((0, N_ACC - n), (0, 0)))

    acc2 = _make_spmm(e_pad, LAT_DIM)(xs2_pad, src, dst, zeros_l)
    out = _tc_final(acc2[0, :n], acc2[1, :n], xs2, cnt0, cnt1,
                    b2.reshape(1, -1))
    return out


# speedup vs baseline: 11.3401x; 11.3401x over previous
import functools

import jax
import jax.numpy as jnp
from jax import lax
from jax.experimental import pallas as pl
from jax.experimental.pallas import tpu as pltpu
from jax.experimental.pallas import tpu_sc as plsc

N_NODES = 10000
IN_DIM = 128
HID_DIM = 128
LAT_DIM = 64

NC, NS = 2, 16
NW = NC * NS
K = 128
N_ACC = 10112
CNT_W = 16
RPS = N_ACC // NS


def _mesh():
    return plsc.VectorSubcoreMesh(core_axis_name="c", subcore_axis_name="s")


_SC_PARAMS = pltpu.CompilerParams(use_tc_tiling_on_sc=False)


@functools.lru_cache(maxsize=None)
def _make_count(e_pad):
    e_w = e_pad // NW
    n_chunks = e_w // K

    @functools.partial(
        pl.kernel,
        out_type=jax.ShapeDtypeStruct((NC, N_ACC, CNT_W), jnp.float32),
        mesh=_mesh(),
        compiler_params=_SC_PARAMS,
        scratch_types=[
            pltpu.VMEM((2, K), jnp.int32),
            pltpu.VMEM((K, CNT_W), jnp.float32),
            pltpu.VMEM_SHARED((N_ACC, CNT_W), jnp.float32),
        ],
    )
    def count_kernel(dst_hbm, ones_hbm, zeros_hbm, out_hbm, idxb, onesb, acc):
        c = lax.axis_index("c")
        s = lax.axis_index("s")
        wid = c * NS + s
        pltpu.sync_copy(ones_hbm, onesb)
        pltpu.sync_copy(zeros_hbm.at[pl.ds(s * RPS, RPS)],
                        acc.at[pl.ds(s * RPS, RPS)])
        plsc.subcore_barrier()

        @pl.loop(0, n_chunks)
        def _(j):
            base = wid * e_w + j * K
            pltpu.sync_copy(dst_hbm.at[pl.ds(base, K)], idxb.at[0])
            pltpu.sync_copy(onesb, acc.at[idxb.at[0]], add=True)

        plsc.subcore_barrier()
        pltpu.sync_copy(acc.at[pl.ds(s * RPS, RPS)],
                        out_hbm.at[c].at[pl.ds(s * RPS, RPS)])

    return count_kernel


@functools.lru_cache(maxsize=None)
def _make_spmm(e_pad, d):
    e_w = e_pad // NW
    n_chunks = e_w // K

    @functools.partial(
        pl.kernel,
        out_type=jax.ShapeDtypeStruct((NC, N_ACC, d), jnp.float32),
        mesh=_mesh(),
        compiler_params=_SC_PARAMS,
        scratch_types=[
            pltpu.VMEM((2, K), jnp.int32),
            pltpu.VMEM((2, K), jnp.int32),
            pltpu.VMEM((K, d), jnp.float32),
            pltpu.VMEM_SHARED((N_ACC, d), jnp.float32),
            pltpu.SemaphoreType.DMA,
        ],
    )
    def spmm_kernel(xs_hbm, src_hbm, dst_hbm, zeros_hbm, out_hbm,
                    srcb, dstb, rowsb, acc, sem):
        c = lax.axis_index("c")
        s = lax.axis_index("s")
        wid = c * NS + s
        pltpu.sync_copy(zeros_hbm.at[pl.ds(s * RPS, RPS)],
                        acc.at[pl.ds(s * RPS, RPS)])
        plsc.subcore_barrier()

        @pl.loop(0, n_chunks)
        def _(j):
            base = wid * e_w + j * K
            pltpu.sync_copy(src_hbm.at[pl.ds(base, K)], srcb.at[0])
            pltpu.sync_copy(dst_hbm.at[pl.ds(base, K)], dstb.at[0])
            pltpu.async_copy(xs_hbm.at[srcb.at[0]], rowsb, sem).wait()
            pltpu.sync_copy(rowsb, acc.at[dstb.at[0]], add=True)

        plsc.subcore_barrier()
        pltpu.sync_copy(acc.at[pl.ds(s * RPS, RPS)],
                        out_hbm.at[c].at[pl.ds(s * RPS, RPS)])

    return spmm_kernel


def _tc_scale_xw(x, w, cnt0, cnt1):
    def body(x_ref, w_ref, c0_ref, c1_ref, o_ref):
        dinv = lax.rsqrt(c0_ref[...] + c1_ref[...] + 1.0)
        xw = jnp.dot(x_ref[...], w_ref[...], preferred_element_type=jnp.float32)
        o_ref[...] = xw * dinv

    return pl.pallas_call(
        body,
        out_shape=jax.ShapeDtypeStruct((x.shape[0], w.shape[1]), jnp.float32),
    )(x, w, cnt0, cnt1)


def _tc_combine_next(a0, a1, xs, cnt0, cnt1, b, w):
    def body(a0_ref, a1_ref, xs_ref, c0_ref, c1_ref, b_ref, w_ref, o_ref):
        dinv = lax.rsqrt(c0_ref[...] + c1_ref[...] + 1.0)
        h = dinv * (a0_ref[...] + a1_ref[...] + xs_ref[...]) + b_ref[...]
        h = jnp.maximum(h, 0.0)
        hw = jnp.dot(h, w_ref[...], preferred_element_type=jnp.float32)
        o_ref[...] = hw * dinv

    return pl.pallas_call(
        body,
        out_shape=jax.ShapeDtypeStruct((a0.shape[0], w.shape[1]), jnp.float32),
    )(a0, a1, xs, cnt0, cnt1, b, w)


def _tc_final(a0, a1, xs, cnt0, cnt1, b):
    def body(a0_ref, a1_ref, xs_ref, c0_ref, c1_ref, b_ref, o_ref):
        dinv = lax.rsqrt(c0_ref[...] + c1_ref[...] + 1.0)
        o_ref[...] = dinv * (a0_ref[...] + a1_ref[...] + xs_ref[...]) + b_ref[...]

    return pl.pallas_call(
        body,
        out_shape=jax.ShapeDtypeStruct(a0.shape, jnp.float32),
    )(a0, a1, xs, cnt0, cnt1, b)


def kernel(x, edge_index, W1, b1, W2, b2):
    n = x.shape[0]
    e = edge_index.shape[1]
    e_pad = ((e + NW * K - 1) // (NW * K)) * (NW * K)

    src = edge_index[0].astype(jnp.int32)
    dst = edge_index[1].astype(jnp.int32)
    pad = jnp.full((e_pad - e,), N_NODES, dtype=jnp.int32)
    src = jnp.concatenate([src, pad])
    dst = jnp.concatenate([dst, pad])

    ones_cnt = jnp.ones((K, CNT_W), jnp.float32)
    zeros_cnt = jnp.zeros((N_ACC, CNT_W), jnp.float32)
    zeros_h = jnp.zeros((N_ACC, HID_DIM), jnp.float32)
    zeros_l = jnp.zeros((N_ACC, LAT_DIM), jnp.float32)

    counts = _make_count(e_pad)(dst, ones_cnt, zeros_cnt)
    cnt0 = counts[0, :n, :1]
    cnt1 = counts[1, :n, :1]

    xs1 = _tc_scale_xw(x, W1, cnt0, cnt1)
    xs1_pad = jnp.pad(xs1, ((0, N_ACC - n), (0, 0)))

    acc1 = _make_spmm(e_pad, HID_DIM)(xs1_pad, src, dst, zeros_h)
    xs2 = _tc_combine_next(acc1[0, :n], acc1[1, :n], xs1, cnt0, cnt1,
                           b1.reshape(1, -1), W2)
    xs2_pad = jnp.pad(xs2, ((0, N_ACC - n), (0, 0)))

    acc2 = _make_spmm(e_pad, LAT_DIM)(xs2_pad, src, dst, zeros_l)
    out = _tc_final(acc2[0, :n], acc2[1, :n], xs2, cnt0, cnt1,
                    b2.reshape(1, -1))
    return out


# speedup vs baseline: 11.6872x; 1.0306x over previous
import functools

import jax
import jax.numpy as jnp
from jax import lax
from jax.experimental import pallas as pl
from jax.experimental.pallas import tpu as pltpu
from jax.experimental.pallas import tpu_sc as plsc

N_NODES = 10000
IN_DIM = 128
HID_DIM = 128
LAT_DIM = 64

NC, NS = 2, 16
NW = NC * NS
K = 128
N_ACC = 10112
CNT_W = 16
RPS = N_ACC // NS


def _mesh():
    return plsc.VectorSubcoreMesh(core_axis_name="c", subcore_axis_name="s",
                                  num_cores=NC, num_subcores=NS)


_SC_PARAMS = pltpu.CompilerParams(use_tc_tiling_on_sc=False)


@functools.lru_cache(maxsize=None)
def _make_count(e_pad, nbuf=8):
    n_chunks = e_pad // NW // K

    @functools.partial(
        pl.kernel,
        out_type=jax.ShapeDtypeStruct((NC, N_ACC, CNT_W), jnp.float32),
        mesh=_mesh(),
        compiler_params=_SC_PARAMS,
        scratch_types=(
            [pltpu.VMEM((n_chunks, K), jnp.int32),
             pltpu.VMEM((K, CNT_W), jnp.float32)]
            + [pltpu.SemaphoreType.DMA] * nbuf
            + [pltpu.VMEM_SHARED((N_ACC, CNT_W), jnp.float32)]
        ),
    )
    def count_kernel(dst_hbm, ones_hbm, zeros_hbm, out_hbm, *refs):
        idxb, onesb = refs[0], refs[1]
        sems = refs[2:2 + nbuf]
        acc = refs[2 + nbuf]
        c = lax.axis_index("c")
        s = lax.axis_index("s")
        wid = c * NS + s
        pltpu.sync_copy(ones_hbm, onesb)
        pltpu.sync_copy(dst_hbm.at[pl.ds(wid * n_chunks, n_chunks)], idxb)
        pltpu.sync_copy(zeros_hbm.at[pl.ds(s * RPS, RPS)],
                        acc.at[pl.ds(s * RPS, RPS)])
        plsc.subcore_barrier()

        @pl.loop(0, n_chunks, step=nbuf)
        def _(j):
            descs = [pltpu.async_copy(onesb, acc.at[idxb.at[j + b]],
                                      sems[b], add=True)
                     for b in range(nbuf)]
            for dsc in descs:
                dsc.wait()

        plsc.subcore_barrier()
        pltpu.sync_copy(acc.at[pl.ds(s * RPS, RPS)],
                        out_hbm.at[c].at[pl.ds(s * RPS, RPS)])

    return count_kernel


@functools.lru_cache(maxsize=None)
def _make_spmm(e_pad, d, nbuf):
    n_chunks = e_pad // NW // K
    n_groups = n_chunks // nbuf

    @functools.partial(
        pl.kernel,
        out_type=jax.ShapeDtypeStruct((NC, N_ACC, d), jnp.float32),
        mesh=_mesh(),
        compiler_params=_SC_PARAMS,
        scratch_types=(
            [pltpu.VMEM((2, nbuf, K), jnp.int32),
             pltpu.VMEM((2, nbuf, K), jnp.int32)]
            + [pltpu.VMEM((K, d), jnp.float32)] * nbuf
            + [pltpu.SemaphoreType.DMA] * (1 + 2 * nbuf)
            + [pltpu.VMEM_SHARED((N_ACC, d), jnp.float32)]
        ),
    )
    def spmm_kernel(xs_hbm, src_hbm, dst_hbm, zeros_hbm, out_hbm, *refs):
        srcb, dstb = refs[0], refs[1]
        rows = refs[2:2 + nbuf]
        isem = refs[2 + nbuf]
        gsems = refs[3 + nbuf:3 + 2 * nbuf]
        ssems = refs[3 + 2 * nbuf:3 + 3 * nbuf]
        acc = refs[3 + 3 * nbuf]
        c = lax.axis_index("c")
        s = lax.axis_index("s")
        wid = c * NS + s
        tbase = wid * n_chunks
        pltpu.sync_copy(src_hbm.at[pl.ds(tbase, nbuf)], srcb.at[0])
        pltpu.sync_copy(dst_hbm.at[pl.ds(tbase, nbuf)], dstb.at[0])
        pltpu.sync_copy(zeros_hbm.at[pl.ds(s * RPS, RPS)],
                        acc.at[pl.ds(s * RPS, RPS)])
        plsc.subcore_barrier()

        @pl.loop(0, n_groups)
        def _(g):
            p = lax.rem(g, 2)
            pn = 1 - p
            nbase = tbase + (g + 1) * nbuf
            id1 = pltpu.async_copy(src_hbm.at[pl.ds(nbase, nbuf)],
                                   srcb.at[pn], isem)
            id2 = pltpu.async_copy(dst_hbm.at[pl.ds(nbase, nbuf)],
                                   dstb.at[pn], isem)
            gds = [pltpu.async_copy(xs_hbm.at[srcb.at[p, b]], rows[b],
                                    gsems[b])
                   for b in range(nbuf)]
            sds = []
            for b in range(nbuf):
                gds[b].wait()
                sds.append(pltpu.async_copy(rows[b], acc.at[dstb.at[p, b]],
                                            ssems[b], add=True))
            for dsc in sds:
                dsc.wait()
            id1.wait()
            id2.wait()

        plsc.subcore_barrier()
        pltpu.sync_copy(acc.at[pl.ds(s * RPS, RPS)],
                        out_hbm.at[c].at[pl.ds(s * RPS, RPS)])

    return spmm_kernel


def _tc_scale_xw(x, w, cnt0, cnt1):
    def body(x_ref, w_ref, c0_ref, c1_ref, o_ref):
        dinv = lax.rsqrt(c0_ref[...] + c1_ref[...] + 1.0)
        xw = jnp.dot(x_ref[...], w_ref[...], preferred_element_type=jnp.float32)
        o_ref[...] = xw * dinv

    return pl.pallas_call(
        body,
        out_shape=jax.ShapeDtypeStruct((x.shape[0], w.shape[1]), jnp.float32),
    )(x, w, cnt0, cnt1)


def _tc_combine_next(a0, a1, xs, cnt0, cnt1, b, w):
    def body(a0_ref, a1_ref, xs_ref, c0_ref, c1_ref, b_ref, w_ref, o_ref):
        dinv = lax.rsqrt(c0_ref[...] + c1_ref[...] + 1.0)
        h = dinv * (a0_ref[...] + a1_ref[...] + xs_ref[...]) + b_ref[...]
        h = jnp.maximum(h, 0.0)
        hw = jnp.dot(h, w_ref[...], preferred_element_type=jnp.float32)
        o_ref[...] = hw * dinv

    return pl.pallas_call(
        body,
        out_shape=jax.ShapeDtypeStruct((a0.shape[0], w.shape[1]), jnp.float32),
    )(a0, a1, xs, cnt0, cnt1, b, w)


def _tc_final(a0, a1, xs, cnt0, cnt1, b):
    def body(a0_ref, a1_ref, xs_ref, c0_ref, c1_ref, b_ref, o_ref):
        dinv = lax.rsqrt(c0_ref[...] + c1_ref[...] + 1.0)
        o_ref[...] = dinv * (a0_ref[...] + a1_ref[...] + xs_ref[...]) + b_ref[...]

    return pl.pallas_call(
        body,
        out_shape=jax.ShapeDtypeStruct(a0.shape, jnp.float32),
    )(a0, a1, xs, cnt0, cnt1, b)


def kernel(x, edge_index, W1, b1, W2, b2):
    n = x.shape[0]
    e = edge_index.shape[1]
    gran = NW * K * 8
    e_pad = ((e + gran - 1) // gran) * gran

    e_rows = e_pad // K + 8
    src = edge_index[0].astype(jnp.int32)
    dst = edge_index[1].astype(jnp.int32)
    pad = jnp.full((e_rows * K - e,), N_NODES, dtype=jnp.int32)
    src = jnp.concatenate([src, pad]).reshape(e_rows, K)
    dst = jnp.concatenate([dst, pad]).reshape(e_rows, K)

    ones_cnt = jnp.ones((K, CNT_W), jnp.float32)
    zeros_cnt = jnp.zeros((N_ACC, CNT_W), jnp.float32)
    zeros_h = jnp.zeros((N_ACC, HID_DIM), jnp.float32)
    zeros_l = jnp.zeros((N_ACC, LAT_DIM), jnp.float32)

    counts = _make_count(e_pad)(dst, ones_cnt, zeros_cnt)
    cnt0 = counts[0, :n, :1]
    cnt1 = counts[1, :n, :1]

    xs1 = _tc_scale_xw(x, W1, cnt0, cnt1)
    xs1_pad = jnp.pad(xs1, ((0, N_ACC - n), (0, 0)))

    acc1 = _make_spmm(e_pad, HID_DIM, 2)(xs1_pad, src, dst, zeros_h)
    xs2 = _tc_combine_next(acc1[0, :n], acc1[1, :n], xs1, cnt0, cnt1,
                           b1.reshape(1, -1), W2)
    xs2_pad = jnp.pad(xs2, ((0, N_ACC - n), (0, 0)))

    acc2 = _make_spmm(e_pad, LAT_DIM, 8)(xs2_pad, src, dst, zeros_l)
    out = _tc_final(acc2[0, :n], acc2[1, :n], xs2, cnt0, cnt1,
                    b2.reshape(1, -1))
    return out


# speedup vs baseline: 12.9651x; 1.1093x over previous
import functools

import jax
import jax.numpy as jnp
from jax import lax
from jax.experimental import pallas as pl
from jax.experimental.pallas import tpu as pltpu
from jax.experimental.pallas import tpu_sc as plsc

N_NODES = 10000
IN_DIM = 128
HID_DIM = 128
LAT_DIM = 64

NC, NS = 2, 16
NW = NC * NS
K = 128
N_ACC = 10112
CNT_W = 16
RPS = N_ACC // NS


def _mesh():
    return plsc.VectorSubcoreMesh(core_axis_name="c", subcore_axis_name="s",
                                  num_cores=NC, num_subcores=NS)


_SC_PARAMS = pltpu.CompilerParams(use_tc_tiling_on_sc=False)


@functools.lru_cache(maxsize=None)
def _make_count(e_pad, nbuf=8):
    n_chunks = e_pad // NW // K

    @functools.partial(
        pl.kernel,
        out_type=jax.ShapeDtypeStruct((NC, N_ACC, CNT_W), jnp.float32),
        mesh=_mesh(),
        compiler_params=_SC_PARAMS,
        scratch_types=(
            [pltpu.VMEM((n_chunks, K), jnp.int32),
             pltpu.VMEM((K, CNT_W), jnp.float32)]
            + [pltpu.SemaphoreType.DMA] * nbuf
            + [pltpu.VMEM_SHARED((N_ACC, CNT_W), jnp.float32)]
        ),
    )
    def count_kernel(dst_hbm, ones_hbm, zeros_hbm, out_hbm, *refs):
        idxb, onesb = refs[0], refs[1]
        sems = refs[2:2 + nbuf]
        acc = refs[2 + nbuf]
        c = lax.axis_index("c")
        s = lax.axis_index("s")
        wid = c * NS + s
        pltpu.sync_copy(ones_hbm, onesb)
        pltpu.sync_copy(dst_hbm.at[pl.ds(wid * n_chunks, n_chunks)], idxb)
        pltpu.sync_copy(zeros_hbm.at[pl.ds(s * RPS, RPS)],
                        acc.at[pl.ds(s * RPS, RPS)])
        plsc.subcore_barrier()

        @pl.loop(0, n_chunks, step=nbuf)
        def _(j):
            descs = [pltpu.async_copy(onesb, acc.at[idxb.at[j + b]],
                                      sems[b], add=True)
                     for b in range(nbuf)]
            for dsc in descs:
                dsc.wait()

        plsc.subcore_barrier()
        pltpu.sync_copy(acc.at[pl.ds(s * RPS, RPS)],
                        out_hbm.at[c].at[pl.ds(s * RPS, RPS)])

    return count_kernel


@functools.lru_cache(maxsize=None)
def _make_spmm(e_pad, d, nbuf, split=None):
    n_chunks = e_pad // NW // K
    n_groups = n_chunks // nbuf
    nc0, nc1 = split if split else (n_chunks, n_chunks)
    assert nc0 % nbuf == 0 and nc1 % nbuf == 0
    assert NS * (nc0 + nc1) == e_pad // K

    @functools.partial(
        pl.kernel,
        out_type=jax.ShapeDtypeStruct((NC, N_ACC, d), jnp.float32),
        mesh=_mesh(),
        compiler_params=_SC_PARAMS,
        scratch_types=(
            [pltpu.VMEM((2, nbuf, K), jnp.int32),
             pltpu.VMEM((2, nbuf, K), jnp.int32)]
            + [pltpu.VMEM((K, d), jnp.float32)] * nbuf
            + [pltpu.SemaphoreType.DMA] * (1 + 2 * nbuf)
            + [pltpu.VMEM_SHARED((N_ACC, d), jnp.float32)]
        ),
    )
    def spmm_kernel(xs_hbm, src_hbm, dst_hbm, zeros_hbm, out_hbm, *refs):
        srcb, dstb = refs[0], refs[1]
        rows = refs[2:2 + nbuf]
        isem = refs[2 + nbuf]
        gsems = refs[3 + nbuf:3 + 2 * nbuf]
        ssems = refs[3 + 2 * nbuf:3 + 3 * nbuf]
        acc = refs[3 + 3 * nbuf]
        c = lax.axis_index("c")
        s = lax.axis_index("s")
        tbase = jnp.where(c == 0, s * nc0, NS * nc0 + s * nc1)
        my_groups = jnp.where(c == 0, nc0 // nbuf, nc1 // nbuf)
        pltpu.sync_copy(src_hbm.at[pl.ds(tbase, nbuf)], srcb.at[0])
        pltpu.sync_copy(dst_hbm.at[pl.ds(tbase, nbuf)], dstb.at[0])
        pltpu.sync_copy(zeros_hbm.at[pl.ds(s * RPS, RPS)],
                        acc.at[pl.ds(s * RPS, RPS)])
        plsc.subcore_barrier()

        @pl.loop(0, my_groups)
        def _(g):
            p = lax.rem(g, 2)
            pn = 1 - p
            nbase = tbase + (g + 1) * nbuf
            id1 = pltpu.async_copy(src_hbm.at[pl.ds(nbase, nbuf)],
                                   srcb.at[pn], isem)
            id2 = pltpu.async_copy(dst_hbm.at[pl.ds(nbase, nbuf)],
                                   dstb.at[pn], isem)
            gds = [pltpu.async_copy(xs_hbm.at[srcb.at[p, b]], rows[b],
                                    gsems[b])
                   for b in range(nbuf)]
            sds = []
            for b in range(nbuf):
                gds[b].wait()
                sds.append(pltpu.async_copy(rows[b], acc.at[dstb.at[p, b]],
                                            ssems[b], add=True))
            for dsc in sds:
                dsc.wait()
            id1.wait()
            id2.wait()

        plsc.subcore_barrier()
        pltpu.sync_copy(acc.at[pl.ds(s * RPS, RPS)],
                        out_hbm.at[c].at[pl.ds(s * RPS, RPS)])

    return spmm_kernel


def _tc_scale_xw(x, w, cnt0, cnt1):
    def body(x_ref, w_ref, c0_ref, c1_ref, o_ref):
        dinv = lax.rsqrt(c0_ref[...] + c1_ref[...] + 1.0)
        xw = jnp.dot(x_ref[...], w_ref[...], preferred_element_type=jnp.float32)
        o_ref[...] = xw * dinv

    return pl.pallas_call(
        body,
        out_shape=jax.ShapeDtypeStruct((x.shape[0], w.shape[1]), jnp.float32),
    )(x, w, cnt0, cnt1)


def _tc_combine_next(a0, a1, xs, cnt0, cnt1, b, w):
    def body(a0_ref, a1_ref, xs_ref, c0_ref, c1_ref, b_ref, w_ref, o_ref):
        dinv = lax.rsqrt(c0_ref[...] + c1_ref[...] + 1.0)
        h = dinv * (a0_ref[...] + a1_ref[...] + xs_ref[...]) + b_ref[...]
        h = jnp.maximum(h, 0.0)
        hw = jnp.dot(h, w_ref[...], preferred_element_type=jnp.float32)
        o_ref[...] = hw * dinv

    return pl.pallas_call(
        body,
        out_shape=jax.ShapeDtypeStruct((a0.shape[0], w.shape[1]), jnp.float32),
    )(a0, a1, xs, cnt0, cnt1, b, w)


def _tc_final(a0, a1, xs, cnt0, cnt1, b):
    def body(a0_ref, a1_ref, xs_ref, c0_ref, c1_ref, b_ref, o_ref):
        dinv = lax.rsqrt(c0_ref[...] + c1_ref[...] + 1.0)
        o_ref[...] = dinv * (a0_ref[...] + a1_ref[...] + xs_ref[...]) + b_ref[...]

    return pl.pallas_call(
        body,
        out_shape=jax.ShapeDtypeStruct(a0.shape, jnp.float32),
    )(a0, a1, xs, cnt0, cnt1, b)


def kernel(x, edge_index, W1, b1, W2, b2):
    n = x.shape[0]
    e = edge_index.shape[1]
    gran = NW * K * 8
    e_pad = ((e + gran - 1) // gran) * gran

    e_rows = e_pad // K + 8
    src = edge_index[0].astype(jnp.int32)
    dst = edge_index[1].astype(jnp.int32)
    pad = jnp.full((e_rows * K - e,), N_NODES, dtype=jnp.int32)
    src = jnp.concatenate([src, pad]).reshape(e_rows, K)
    dst = jnp.concatenate([dst, pad]).reshape(e_rows, K)

    ones_cnt = jnp.ones((K, CNT_W), jnp.float32)
    zeros_cnt = jnp.zeros((N_ACC, CNT_W), jnp.float32)
    zeros_h = jnp.zeros((N_ACC, HID_DIM), jnp.float32)
    zeros_l = jnp.zeros((N_ACC, LAT_DIM), jnp.float32)

    counts = _make_count(e_pad)(dst, ones_cnt, zeros_cnt)
    cnt0 = counts[0, :n, :1]
    cnt1 = counts[1, :n, :1]

    xs1 = _tc_scale_xw(x, W1, cnt0, cnt1)
    xs1_pad = jnp.pad(xs1, ((0, N_ACC - n), (0, 0)))

    acc1 = _make_spmm(e_pad, HID_DIM, 2, (120, 40))(xs1_pad, src, dst, zeros_h)
    xs2 = _tc_combine_next(acc1[0, :n], acc1[1, :n], xs1, cnt0, cnt1,
                           b1.reshape(1, -1), W2)
    xs2_pad = jnp.pad(xs2, ((0, N_ACC - n), (0, 0)))

    acc2 = _make_spmm(e_pad, LAT_DIM, 8, (120, 40))(xs2_pad, src, dst, zeros_l)
    out = _tc_final(acc2[0, :n], acc2[1, :n], xs2, cnt0, cnt1,
                    b2.reshape(1, -1))
    return out
